# baseline (speedup 1.0000x reference)
import dataclasses
import functools

import jax
import jax.numpy as jnp
from jax import lax
from jax.experimental import pallas as pl
from jax.experimental.pallas import tpu as pltpu
from jax.experimental.pallas import tpu_sc as plsc

B = 16384
F = 18
V = 1000000
EMB = 16
DE = 256
BF = B * F

NC = 2
NS = 16
NW = NC * NS
B_PER_W = BF // NW
CHUNK = 32


FW_ROWS = 7813
KB = 64
EW = 384


def _sc_gather(emb_tbl, fw128, idx, frow, flane):
    mesh = plsc.VectorSubcoreMesh(core_axis_name="c", subcore_axis_name="s")
    cp = pltpu.CompilerParams()
    if "needs_layout_passes" in pltpu.CompilerParams.__dataclass_fields__:
        cp = dataclasses.replace(cp, needs_layout_passes=False)

    @functools.partial(
        pl.kernel,
        compiler_params=cp,
        out_type=(
            jax.ShapeDtypeStruct((BF, EMB), jnp.float32),
            jax.ShapeDtypeStruct((BF,), jnp.float32),
        ),
        mesh=mesh,
        scratch_types=[
            pltpu.VMEM((B_PER_W,), jnp.int32),
            pltpu.VMEM((B_PER_W,), jnp.int32),
            pltpu.VMEM((B_PER_W,), jnp.int32),
            pltpu.VMEM((KB, 128), jnp.float32),
            pltpu.VMEM((KB,), jnp.float32),
            pltpu.SemaphoreType.DMA,
            pltpu.SemaphoreType.DMA,
        ],
    )
    def k(emb_hbm, fw_hbm, idx_hbm, frow_hbm, flane_hbm,
          emb_out, fw_out,
          idx_v, frow_v, flane_v, fbuf, fv, sem_e, sem_f):
        wid = lax.axis_index("s") * NC + lax.axis_index("c")
        base = wid * B_PER_W
        pltpu.sync_copy(idx_hbm.at[pl.ds(base, B_PER_W)], idx_v)
        pltpu.sync_copy(frow_hbm.at[pl.ds(base, B_PER_W)], frow_v)
        pltpu.sync_copy(flane_hbm.at[pl.ds(base, B_PER_W)], flane_v)

        @pl.loop(0, B_PER_W, step=KB)
        def _(c):
            cp_f = pltpu.async_copy(fw_hbm.at[frow_v.at[pl.ds(c, KB)]],
                                    fbuf, sem_f)
            copies = []
            for g in range(KB // 16):
                vec = idx_v[pl.ds(c + g * 16, 16)]
                for j in range(16):
                    kflat = base + c + g * 16 + j
                    copies.append(pltpu.async_copy(
                        emb_hbm.at[pl.ds(vec[j], 1)],
                        emb_out.at[pl.ds(kflat, 1)],
                        sem_e))
            cp_f.wait()

            @pl.loop(0, KB // 16)
            def _(s):
                rows = lax.broadcasted_iota(jnp.int32, (16,), 0) + s * 16
                lanes = flane_v[pl.ds(c + s * 16, 16)]
                fv[pl.ds(s * 16, 16)] = plsc.load_gather(fbuf, [rows, lanes])

            pltpu.sync_copy(fv, fw_out.at[pl.ds(base + c, KB)])
            for cp_e in copies:
                cp_e.wait()

    return k(emb_tbl, fw128, idx, frow, flane)


BLK = 1024


def _tc_body(emb_ref, val_ref, fw_ref, fb_ref,
             w1_ref, b1_ref, w2_ref, b2_ref, w3_ref, b3_ref,
             g1_ref, be1_ref, m1_ref, v1_ref,
             g2_ref, be2_ref, m2_ref, v2_ref,
             g3_ref, be3_ref, m3_ref, v3_ref,
             out_ref):
    emb = emb_ref[...]
    val = val_ref[...]
    fw = fw_ref[...]

    col_f = lax.broadcasted_iota(jnp.int32, (F, F * EMB), 1) // EMB
    row_f = lax.broadcasted_iota(jnp.int32, (F, F * EMB), 0)
    E = (col_f == row_f).astype(jnp.float32)

    vexp = jnp.dot(val, E, preferred_element_type=jnp.float32)
    sw = vexp * emb

    Gt = E.T
    s1 = jnp.dot(sw, Gt, preferred_element_type=jnp.float32)
    s2 = jnp.dot(sw * sw, Gt, preferred_element_type=jnp.float32)
    second = 0.5 * jnp.sum(s1 * s1 - s2, axis=1, keepdims=True)

    first = jnp.sum(fw * val, axis=1, keepdims=True) + fb_ref[0, 0]

    def bn(x, g_ref, be_ref, m_ref, v_ref):
        return (x - m_ref[...]) * lax.rsqrt(v_ref[...] + 1e-3) * g_ref[...] + be_ref[...]

    a = jnp.dot(sw, w1_ref[...], preferred_element_type=jnp.float32) + b1_ref[...]
    a = jnp.maximum(bn(a, g1_ref, be1_ref, m1_ref, v1_ref), 0.0)
    a = jnp.dot(a, w2_ref[...], preferred_element_type=jnp.float32) + b2_ref[...]
    a = jnp.maximum(bn(a, g2_ref, be2_ref, m2_ref, v2_ref), 0.0)
    a = jnp.dot(a, w3_ref[...], preferred_element_type=jnp.float32) + b3_ref[...]
    a = bn(a, g3_ref, be3_ref, m3_ref, v3_ref)

    out_ref[:, 0:1] = first
    out_ref[:, 1:2] = second
    out_ref[:, 2:] = a


def _row_spec(n_cols):
    return pl.BlockSpec((BLK, n_cols), lambda i: (i, 0))


def _full_spec(shape):
    return pl.BlockSpec(shape, lambda i: tuple(0 for _ in shape))


def kernel(feature_index, feature_value, feature_weight, first_weight, first_bias,
           dense1, bias1, dense2, bias2, dense3, bias3,
           bn1_gamma, bn1_beta, bn1_mean, bn1_var,
           bn2_gamma, bn2_beta, bn2_mean, bn2_var,
           bn3_gamma, bn3_beta, bn3_mean, bn3_var):
    idx_flat = feature_index.astype(jnp.int32).reshape(BF)
    fw128 = jnp.concatenate(
        [first_weight,
         jnp.zeros((FW_ROWS * 128 - V,), jnp.float32)]).reshape(FW_ROWS, 128)
    emb_g, fw_g = _sc_gather(
        feature_weight, fw128, idx_flat,
        idx_flat // 128, idx_flat % 128)
    emb2d = emb_g.reshape(B, F * EMB)
    fw2d = fw_g.reshape(B, F)

    row1 = lambda x: x.reshape(1, -1)
    args = (emb2d, feature_value, fw2d, first_bias.reshape(1, 1),
            dense1, row1(bias1), dense2, row1(bias2), dense3, row1(bias3),
            row1(bn1_gamma), row1(bn1_beta), row1(bn1_mean), row1(bn1_var),
            row1(bn2_gamma), row1(bn2_beta), row1(bn2_mean), row1(bn2_var),
            row1(bn3_gamma), row1(bn3_beta), row1(bn3_mean), row1(bn3_var))

    in_specs = [
        _row_spec(F * EMB), _row_spec(F), _row_spec(F), _full_spec((1, 1)),
        _full_spec((F * EMB, DE)), _full_spec((1, DE)),
        _full_spec((DE, DE // 2)), _full_spec((1, DE // 2)),
        _full_spec((DE // 2, DE // 4)), _full_spec((1, DE // 4)),
    ] + [_full_spec((1, DE))] * 4 \
      + [_full_spec((1, DE // 2))] * 4 \
      + [_full_spec((1, DE // 4))] * 4

    out = pl.pallas_call(
        _tc_body,
        grid=(B // BLK,),
        in_specs=in_specs,
        out_specs=pl.BlockSpec((BLK, 2 + DE // 4), lambda i: (i, 0)),
        out_shape=jax.ShapeDtypeStruct((B, 2 + DE // 4), jnp.float32),
    )(*args)
    return out


# speedup vs baseline: 6.7348x; 6.7348x over previous
import dataclasses
import functools

import jax
import jax.numpy as jnp
from jax import lax
from jax.experimental import pallas as pl
from jax.experimental.pallas import tpu as pltpu
from jax.experimental.pallas import tpu_sc as plsc

B = 16384
F = 18
V = 1000000
EMB = 16
DE = 256
BF = B * F

NC = 2
NS = 16
NW = NC * NS
B_PER_W = BF // NW
CHUNK = 128

FW_ROWS = 7813


def _sc_gather(emb128, fw128, idx):
    mesh = plsc.VectorSubcoreMesh(core_axis_name="c", subcore_axis_name="s")
    cp = pltpu.CompilerParams()
    if "needs_layout_passes" in pltpu.CompilerParams.__dataclass_fields__:
        cp = dataclasses.replace(cp, needs_layout_passes=False)

    @functools.partial(
        pl.kernel,
        compiler_params=cp,
        out_type=(
            jax.ShapeDtypeStruct((BF // 8, 128), jnp.float32),
            jax.ShapeDtypeStruct((BF,), jnp.float32),
        ),
        mesh=mesh,
        scratch_types=[
            pltpu.VMEM((B_PER_W,), jnp.int32),
            pltpu.VMEM((CHUNK,), jnp.int32),
            pltpu.VMEM((CHUNK,), jnp.int32),
            pltpu.VMEM((CHUNK,), jnp.int32),
            pltpu.VMEM((CHUNK,), jnp.int32),
            pltpu.VMEM((CHUNK, 128), jnp.float32),
            pltpu.VMEM((CHUNK, 128), jnp.float32),
            pltpu.VMEM((CHUNK // 8, 128), jnp.float32),
            pltpu.VMEM((CHUNK,), jnp.float32),
            pltpu.SemaphoreType.DMA,
            pltpu.SemaphoreType.DMA,
        ],
    )
    def k(emb_hbm, fw_hbm, idx_hbm,
          emb_out, fw_out,
          idx_v, erow_v, elane_v, frow_v, flane_v, ebuf, fbuf, e2, fv,
          sem_e, sem_f):
        wid = lax.axis_index("s") * NC + lax.axis_index("c")
        base = wid * B_PER_W
        pltpu.sync_copy(idx_hbm.at[pl.ds(base, B_PER_W)], idx_v)

        @pl.loop(0, B_PER_W, step=CHUNK)
        def _(c):
            @pl.loop(0, CHUNK, step=16)
            def _(g):
                ix = idx_v[pl.ds(c + g, 16)]
                erow_v[pl.ds(g, 16)] = lax.shift_right_logical(ix, 3)
                elane_v[pl.ds(g, 16)] = lax.bitwise_and(ix, 7) * EMB
                frow_v[pl.ds(g, 16)] = lax.shift_right_logical(ix, 7)
                flane_v[pl.ds(g, 16)] = lax.bitwise_and(ix, 127)

            cp_e = pltpu.async_copy(emb_hbm.at[erow_v], ebuf, sem_e)
            cp_f = pltpu.async_copy(fw_hbm.at[frow_v], fbuf, sem_f)
            cp_e.wait()

            @pl.loop(0, CHUNK, step=16)
            def _(c16):
                lbv = elane_v[pl.ds(c16, 16)]
                for j in range(16):
                    e2.at[c16 // 8 + j // 8,
                          pl.ds((j % 8) * EMB, EMB)][...] = (
                        ebuf.at[c16 + j, pl.ds(lbv[j], EMB)][...])

            cp_f.wait()

            @pl.loop(0, CHUNK, step=16)
            def _(s16):
                rows = lax.broadcasted_iota(jnp.int32, (16,), 0) + s16
                lanes = flane_v[pl.ds(s16, 16)]
                fv[pl.ds(s16, 16)] = plsc.load_gather(fbuf, [rows, lanes])

            orow = pl.multiple_of((base + c) // 8, 8)
            pltpu.sync_copy(e2, emb_out.at[pl.ds(orow, CHUNK // 8)])
            pltpu.sync_copy(fv, fw_out.at[pl.ds(base + c, CHUNK)])

    return k(emb128, fw128, idx)


BLK = 1024


def _tc_body(emb_ref, val_ref, fw_ref, fb_ref,
             w1_ref, b1_ref, w2_ref, b2_ref, w3_ref, b3_ref,
             g1_ref, be1_ref, m1_ref, v1_ref,
             g2_ref, be2_ref, m2_ref, v2_ref,
             g3_ref, be3_ref, m3_ref, v3_ref,
             out_ref):
    emb = emb_ref[...]
    val = val_ref[...]
    fw = fw_ref[...]

    col_f = lax.broadcasted_iota(jnp.int32, (F, F * EMB), 1) // EMB
    row_f = lax.broadcasted_iota(jnp.int32, (F, F * EMB), 0)
    E = (col_f == row_f).astype(jnp.float32)

    vexp = jnp.dot(val, E, preferred_element_type=jnp.float32)
    sw = vexp * emb

    Gt = E.T
    s1 = jnp.dot(sw, Gt, preferred_element_type=jnp.float32)
    s2 = jnp.dot(sw * sw, Gt, preferred_element_type=jnp.float32)
    second = 0.5 * jnp.sum(s1 * s1 - s2, axis=1, keepdims=True)

    first = jnp.sum(fw * val, axis=1, keepdims=True) + fb_ref[0, 0]

    def bn(x, g_ref, be_ref, m_ref, v_ref):
        return (x - m_ref[...]) * lax.rsqrt(v_ref[...] + 1e-3) * g_ref[...] + be_ref[...]

    a = jnp.dot(sw, w1_ref[...], preferred_element_type=jnp.float32) + b1_ref[...]
    a = jnp.maximum(bn(a, g1_ref, be1_ref, m1_ref, v1_ref), 0.0)
    a = jnp.dot(a, w2_ref[...], preferred_element_type=jnp.float32) + b2_ref[...]
    a = jnp.maximum(bn(a, g2_ref, be2_ref, m2_ref, v2_ref), 0.0)
    a = jnp.dot(a, w3_ref[...], preferred_element_type=jnp.float32) + b3_ref[...]
    a = bn(a, g3_ref, be3_ref, m3_ref, v3_ref)

    out_ref[:, 0:1] = first
    out_ref[:, 1:2] = second
    out_ref[:, 2:] = a


def _row_spec(n_cols):
    return pl.BlockSpec((BLK, n_cols), lambda i: (i, 0))


def _full_spec(shape):
    return pl.BlockSpec(shape, lambda i: tuple(0 for _ in shape))


def kernel(feature_index, feature_value, feature_weight, first_weight, first_bias,
           dense1, bias1, dense2, bias2, dense3, bias3,
           bn1_gamma, bn1_beta, bn1_mean, bn1_var,
           bn2_gamma, bn2_beta, bn2_mean, bn2_var,
           bn3_gamma, bn3_beta, bn3_mean, bn3_var):
    idx_flat = feature_index.astype(jnp.int32).reshape(BF)
    emb128 = feature_weight.reshape(-1).reshape(V // 8, 128)
    fw_pad = jnp.concatenate(
        [first_weight, jnp.zeros((FW_ROWS * 128 - V,), jnp.float32)]
    ).reshape(FW_ROWS, 128)
    emb_g, fw_g = _sc_gather(emb128, fw_pad, idx_flat)

    emb2d = emb_g.reshape(B, F * EMB)
    fw2d = fw_g.reshape(B, F)

    row1 = lambda x: x.reshape(1, -1)
    args = (emb2d, feature_value, fw2d, first_bias.reshape(1, 1),
            dense1, row1(bias1), dense2, row1(bias2), dense3, row1(bias3),
            row1(bn1_gamma), row1(bn1_beta), row1(bn1_mean), row1(bn1_var),
            row1(bn2_gamma), row1(bn2_beta), row1(bn2_mean), row1(bn2_var),
            row1(bn3_gamma), row1(bn3_beta), row1(bn3_mean), row1(bn3_var))

    in_specs = [
        _row_spec(F * EMB), _row_spec(F), _row_spec(F), _full_spec((1, 1)),
        _full_spec((F * EMB, DE)), _full_spec((1, DE)),
        _full_spec((DE, DE // 2)), _full_spec((1, DE // 2)),
        _full_spec((DE // 2, DE // 4)), _full_spec((1, DE // 4)),
    ] + [_full_spec((1, DE))] * 4 \
      + [_full_spec((1, DE // 2))] * 4 \
      + [_full_spec((1, DE // 4))] * 4

    out = pl.pallas_call(
        _tc_body,
        grid=(B // BLK,),
        in_specs=in_specs,
        out_specs=pl.BlockSpec((BLK, 2 + DE // 4), lambda i: (i, 0)),
        out_shape=jax.ShapeDtypeStruct((B, 2 + DE // 4), jnp.float32),
    )(*args)
    return out


# speedup vs baseline: 7.1433x; 1.0607x over previous
import dataclasses
import functools

import jax
import jax.numpy as jnp
from jax import lax
from jax.experimental import pallas as pl
from jax.experimental.pallas import tpu as pltpu
from jax.experimental.pallas import tpu_sc as plsc

B = 16384
F = 18
V = 1000000
EMB = 16
DE = 256
BF = B * F

NC = 2
NS = 16
NW = NC * NS
B_PER_W = BF // NW
CHUNK = 128

def _sc_gather(emb128, fw1d, idx):
    mesh = plsc.VectorSubcoreMesh(core_axis_name="c", subcore_axis_name="s")
    cp = pltpu.CompilerParams()
    if "needs_layout_passes" in pltpu.CompilerParams.__dataclass_fields__:
        cp = dataclasses.replace(cp, needs_layout_passes=False)

    @functools.partial(
        pl.kernel,
        compiler_params=cp,
        out_type=(
            jax.ShapeDtypeStruct((BF // 8, 128), jnp.float32),
            jax.ShapeDtypeStruct((BF,), jnp.float32),
        ),
        mesh=mesh,
        scratch_types=[
            pltpu.VMEM((B_PER_W,), jnp.int32),
            pltpu.VMEM((CHUNK,), jnp.int32),
            pltpu.VMEM((CHUNK,), jnp.int32),
            pltpu.VMEM((CHUNK, 128), jnp.float32),
            pltpu.VMEM((CHUNK // 8, 128), jnp.float32),
            pltpu.VMEM((CHUNK,), jnp.float32),
            pltpu.SemaphoreType.DMA,
            pltpu.SemaphoreType.DMA,
        ],
    )
    def k(emb_hbm, fw_hbm, idx_hbm,
          emb_out, fw_out,
          idx_v, erow_v, elane_v, ebuf, e2, fv,
          sem_e, sem_f):
        wid = lax.axis_index("s") * NC + lax.axis_index("c")
        base = wid * B_PER_W
        pltpu.sync_copy(idx_hbm.at[pl.ds(base, B_PER_W)], idx_v)

        @pl.loop(0, B_PER_W, step=CHUNK)
        def _(c):
            @pl.loop(0, CHUNK, step=16)
            def _(g):
                ix = idx_v[pl.ds(c + g, 16)]
                erow_v[pl.ds(g, 16)] = lax.shift_right_logical(ix, 3)
                elane_v[pl.ds(g, 16)] = lax.bitwise_and(ix, 7) * EMB

            cp_e = pltpu.async_copy(emb_hbm.at[erow_v], ebuf, sem_e)
            cp_f = pltpu.async_copy(fw_hbm.at[idx_v.at[pl.ds(c, CHUNK)]],
                                    fv, sem_f)
            cp_e.wait()

            @pl.loop(0, CHUNK, step=16)
            def _(c16):
                lbv = elane_v[pl.ds(c16, 16)]
                for j in range(16):
                    e2.at[c16 // 8 + j // 8,
                          pl.ds((j % 8) * EMB, EMB)][...] = (
                        ebuf.at[c16 + j, pl.ds(lbv[j], EMB)][...])

            cp_f.wait()
            orow = pl.multiple_of((base + c) // 8, 8)
            pltpu.sync_copy(e2, emb_out.at[pl.ds(orow, CHUNK // 8)])
            pltpu.sync_copy(fv, fw_out.at[pl.ds(base + c, CHUNK)])

    return k(emb128, fw1d, idx)


BLK = 1024


def _tc_body(emb_ref, val_ref, fw_ref, fb_ref,
             w1_ref, b1_ref, w2_ref, b2_ref, w3_ref, b3_ref,
             g1_ref, be1_ref, m1_ref, v1_ref,
             g2_ref, be2_ref, m2_ref, v2_ref,
             g3_ref, be3_ref, m3_ref, v3_ref,
             out_ref):
    emb = emb_ref[...]
    val = val_ref[...]
    fw = fw_ref[...]

    col_f = lax.broadcasted_iota(jnp.int32, (F, F * EMB), 1) // EMB
    row_f = lax.broadcasted_iota(jnp.int32, (F, F * EMB), 0)
    E = (col_f == row_f).astype(jnp.float32)

    vexp = jnp.dot(val, E, preferred_element_type=jnp.float32)
    sw = vexp * emb

    Gt = E.T
    s1 = jnp.dot(sw, Gt, preferred_element_type=jnp.float32)
    s2 = jnp.dot(sw * sw, Gt, preferred_element_type=jnp.float32)
    second = 0.5 * jnp.sum(s1 * s1 - s2, axis=1, keepdims=True)

    first = jnp.sum(fw * val, axis=1, keepdims=True) + fb_ref[0, 0]

    def bn(x, g_ref, be_ref, m_ref, v_ref):
        return (x - m_ref[...]) * lax.rsqrt(v_ref[...] + 1e-3) * g_ref[...] + be_ref[...]

    a = jnp.dot(sw, w1_ref[...], preferred_element_type=jnp.float32) + b1_ref[...]
    a = jnp.maximum(bn(a, g1_ref, be1_ref, m1_ref, v1_ref), 0.0)
    a = jnp.dot(a, w2_ref[...], preferred_element_type=jnp.float32) + b2_ref[...]
    a = jnp.maximum(bn(a, g2_ref, be2_ref, m2_ref, v2_ref), 0.0)
    a = jnp.dot(a, w3_ref[...], preferred_element_type=jnp.float32) + b3_ref[...]
    a = bn(a, g3_ref, be3_ref, m3_ref, v3_ref)

    out_ref[:, 0:1] = first
    out_ref[:, 1:2] = second
    out_ref[:, 2:] = a


def _row_spec(n_cols):
    return pl.BlockSpec((BLK, n_cols), lambda i: (i, 0))


def _full_spec(shape):
    return pl.BlockSpec(shape, lambda i: tuple(0 for _ in shape))


def kernel(feature_index, feature_value, feature_weight, first_weight, first_bias,
           dense1, bias1, dense2, bias2, dense3, bias3,
           bn1_gamma, bn1_beta, bn1_mean, bn1_var,
           bn2_gamma, bn2_beta, bn2_mean, bn2_var,
           bn3_gamma, bn3_beta, bn3_mean, bn3_var):
    idx_flat = feature_index.astype(jnp.int32).reshape(BF)
    emb128 = feature_weight.reshape(-1).reshape(V // 8, 128)
    emb_g, fw_g = _sc_gather(emb128, first_weight, idx_flat)

    emb2d = emb_g.reshape(B, F * EMB)
    fw2d = fw_g.reshape(B, F)

    row1 = lambda x: x.reshape(1, -1)
    args = (emb2d, feature_value, fw2d, first_bias.reshape(1, 1),
            dense1, row1(bias1), dense2, row1(bias2), dense3, row1(bias3),
            row1(bn1_gamma), row1(bn1_beta), row1(bn1_mean), row1(bn1_var),
            row1(bn2_gamma), row1(bn2_beta), row1(bn2_mean), row1(bn2_var),
            row1(bn3_gamma), row1(bn3_beta), row1(bn3_mean), row1(bn3_var))

    in_specs = [
        _row_spec(F * EMB), _row_spec(F), _row_spec(F), _full_spec((1, 1)),
        _full_spec((F * EMB, DE)), _full_spec((1, DE)),
        _full_spec((DE, DE // 2)), _full_spec((1, DE // 2)),
        _full_spec((DE // 2, DE // 4)), _full_spec((1, DE // 4)),
    ] + [_full_spec((1, DE))] * 4 \
      + [_full_spec((1, DE // 2))] * 4 \
      + [_full_spec((1, DE // 4))] * 4

    out = pl.pallas_call(
        _tc_body,
        grid=(B // BLK,),
        in_specs=in_specs,
        out_specs=pl.BlockSpec((BLK, 2 + DE // 4), lambda i: (i, 0)),
        out_shape=jax.ShapeDtypeStruct((B, 2 + DE // 4), jnp.float32),
    )(*args)
    return out


# speedup vs baseline: 7.1528x; 1.0013x over previous
import dataclasses
import functools

import jax
import jax.numpy as jnp
from jax import lax
from jax.experimental import pallas as pl
from jax.experimental.pallas import tpu as pltpu
from jax.experimental.pallas import tpu_sc as plsc

B = 16384
F = 18
V = 1000000
EMB = 16
DE = 256
BF = B * F

NC = 2
NS = 16
NW = NC * NS
B_PER_W = BF // NW
CHUNK = 128

def _sc_gather(emb128, fw1d, idx):
    mesh = plsc.VectorSubcoreMesh(core_axis_name="c", subcore_axis_name="s")
    cp = pltpu.CompilerParams()
    if "needs_layout_passes" in pltpu.CompilerParams.__dataclass_fields__:
        cp = dataclasses.replace(cp, needs_layout_passes=False)

    @functools.partial(
        pl.kernel,
        compiler_params=cp,
        out_type=(
            jax.ShapeDtypeStruct((BF // 8, 128), jnp.float32),
            jax.ShapeDtypeStruct((BF,), jnp.float32),
        ),
        mesh=mesh,
        scratch_types=[
            pltpu.VMEM((B_PER_W // 128, 128), jnp.int32),
            pltpu.VMEM((CHUNK,), jnp.int32),
            pltpu.VMEM((CHUNK,), jnp.int32),
            pltpu.VMEM((CHUNK,), jnp.int32),
            pltpu.VMEM((CHUNK, 128), jnp.float32),
            pltpu.VMEM((CHUNK // 8, 128), jnp.float32),
            pltpu.VMEM((CHUNK,), jnp.float32),
            pltpu.SemaphoreType.DMA,
            pltpu.SemaphoreType.DMA,
        ],
    )
    def k(emb_hbm, fw_hbm, idx_hbm,
          emb_out, fw_out,
          idx_v, idxc_v, erow_v, elane_v, ebuf, e2, fv,
          sem_e, sem_f):
        wid = lax.axis_index("s") * NC + lax.axis_index("c")
        base = wid * B_PER_W
        rpw = B_PER_W // 128
        pltpu.sync_copy(idx_hbm.at[pl.ds(wid * rpw, rpw)], idx_v)

        @pl.loop(0, B_PER_W, step=CHUNK)
        def _(c):
            @pl.loop(0, CHUNK, step=16)
            def _(g):
                ix = idx_v[c // 128, pl.ds(g, 16)]
                idxc_v[pl.ds(g, 16)] = ix
                erow_v[pl.ds(g, 16)] = lax.shift_right_logical(ix, 3)
                elane_v[pl.ds(g, 16)] = lax.bitwise_and(ix, 7) * EMB

            cp_e = pltpu.async_copy(emb_hbm.at[erow_v], ebuf, sem_e)
            cp_f = pltpu.async_copy(fw_hbm.at[idxc_v], fv, sem_f)
            cp_e.wait()

            @pl.loop(0, CHUNK, step=16)
            def _(c16):
                lbv = elane_v[pl.ds(c16, 16)]
                for j in range(16):
                    e2.at[c16 // 8 + j // 8,
                          pl.ds((j % 8) * EMB, EMB)][...] = (
                        ebuf.at[c16 + j, pl.ds(lbv[j], EMB)][...])

            cp_f.wait()
            orow = pl.multiple_of((base + c) // 8, 8)
            pltpu.sync_copy(e2, emb_out.at[pl.ds(orow, CHUNK // 8)])
            pltpu.sync_copy(fv, fw_out.at[pl.ds(base + c, CHUNK)])

    return k(emb128, fw1d, idx)


BLK = 1024


def _tc_body(emb_ref, val_ref, fw_ref, fb_ref,
             w1_ref, b1_ref, w2_ref, b2_ref, w3_ref, b3_ref,
             g1_ref, be1_ref, m1_ref, v1_ref,
             g2_ref, be2_ref, m2_ref, v2_ref,
             g3_ref, be3_ref, m3_ref, v3_ref,
             out_ref):
    emb = emb_ref[...]
    val = val_ref[...]
    fw = fw_ref[...]

    col_f = lax.broadcasted_iota(jnp.int32, (F, F * EMB), 1) // EMB
    row_f = lax.broadcasted_iota(jnp.int32, (F, F * EMB), 0)
    E = (col_f == row_f).astype(jnp.float32)

    vexp = jnp.dot(val, E, preferred_element_type=jnp.float32)
    sw = vexp * emb

    Gt = E.T
    s1 = jnp.dot(sw, Gt, preferred_element_type=jnp.float32)
    s2 = jnp.dot(sw * sw, Gt, preferred_element_type=jnp.float32)
    second = 0.5 * jnp.sum(s1 * s1 - s2, axis=1, keepdims=True)

    first = jnp.sum(fw * val, axis=1, keepdims=True) + fb_ref[0, 0]

    def bn(x, g_ref, be_ref, m_ref, v_ref):
        return (x - m_ref[...]) * lax.rsqrt(v_ref[...] + 1e-3) * g_ref[...] + be_ref[...]

    a = jnp.dot(sw, w1_ref[...], preferred_element_type=jnp.float32) + b1_ref[...]
    a = jnp.maximum(bn(a, g1_ref, be1_ref, m1_ref, v1_ref), 0.0)
    a = jnp.dot(a, w2_ref[...], preferred_element_type=jnp.float32) + b2_ref[...]
    a = jnp.maximum(bn(a, g2_ref, be2_ref, m2_ref, v2_ref), 0.0)
    a = jnp.dot(a, w3_ref[...], preferred_element_type=jnp.float32) + b3_ref[...]
    a = bn(a, g3_ref, be3_ref, m3_ref, v3_ref)

    out_ref[:, 0:1] = first
    out_ref[:, 1:2] = second
    out_ref[:, 2:] = a


def _row_spec(n_cols):
    return pl.BlockSpec((BLK, n_cols), lambda i: (i, 0))


def _full_spec(shape):
    return pl.BlockSpec(shape, lambda i: tuple(0 for _ in shape))


def kernel(feature_index, feature_value, feature_weight, first_weight, first_bias,
           dense1, bias1, dense2, bias2, dense3, bias3,
           bn1_gamma, bn1_beta, bn1_mean, bn1_var,
           bn2_gamma, bn2_beta, bn2_mean, bn2_var,
           bn3_gamma, bn3_beta, bn3_mean, bn3_var):
    idx2d = feature_index.astype(jnp.int32).reshape(BF // 128, 128)
    emb128 = feature_weight.reshape(-1).reshape(V // 8, 128)
    emb_g, fw_g = _sc_gather(emb128, first_weight, idx2d)

    emb2d = emb_g.reshape(B, F * EMB)
    fw2d = fw_g.reshape(B, F)

    row1 = lambda x: x.reshape(1, -1)
    args = (emb2d, feature_value, fw2d, first_bias.reshape(1, 1),
            dense1, row1(bias1), dense2, row1(bias2), dense3, row1(bias3),
            row1(bn1_gamma), row1(bn1_beta), row1(bn1_mean), row1(bn1_var),
            row1(bn2_gamma), row1(bn2_beta), row1(bn2_mean), row1(bn2_var),
            row1(bn3_gamma), row1(bn3_beta), row1(bn3_mean), row1(bn3_var))

    in_specs = [
        _row_spec(F * EMB), _row_spec(F), _row_spec(F), _full_spec((1, 1)),
        _full_spec((F * EMB, DE)), _full_spec((1, DE)),
        _full_spec((DE, DE // 2)), _full_spec((1, DE // 2)),
        _full_spec((DE // 2, DE // 4)), _full_spec((1, DE // 4)),
    ] + [_full_spec((1, DE))] * 4 \
      + [_full_spec((1, DE // 2))] * 4 \
      + [_full_spec((1, DE // 4))] * 4

    out = pl.pallas_call(
        _tc_body,
        grid=(B // BLK,),
        in_specs=in_specs,
        out_specs=pl.BlockSpec((BLK, 2 + DE // 4), lambda i: (i, 0)),
        out_shape=jax.ShapeDtypeStruct((B, 2 + DE // 4), jnp.float32),
    )(*args)
    return out
